# baseline (speedup 1.0000x reference)
import functools
import math

import jax
import jax.numpy as jnp
from jax import lax
from jax.experimental import pallas as pl
from jax.experimental.pallas import tpu as pltpu
from jax.experimental.pallas import tpu_sc as plsc

C_DIM = 32
SCALE = 1.0 / math.sqrt(C_DIM)


@jax.jit
def _emb_lookup(tag_t, table):
    S, B = tag_t.shape
    NBT = B // 128
    NCT = C_DIM // 8
    info = plsc.get_sparse_core_info()
    nw = info.num_cores * info.num_subcores
    n_units = S * NBT
    units_per_w = n_units // nw
    n_pairs = units_per_w // 2

    mesh = plsc.VectorSubcoreMesh(core_axis_name="c", subcore_axis_name="s")

    @functools.partial(
        pl.kernel,
        mesh=mesh,
        out_type=jax.ShapeDtypeStruct((S, NCT, NBT, 8, 128), jnp.float32),
        scratch_types=[
            pltpu.VMEM((2, 128), jnp.int32),
            pltpu.VMEM((2, 128), jnp.int32),
            pltpu.VMEM((2, 128, C_DIM // 2), jnp.int32),
            pltpu.VMEM((2, NCT, 8, 137), jnp.float32),
            pltpu.SemaphoreType.DMA,
            pltpu.SemaphoreType.DMA,
            pltpu.SemaphoreType.DMA,
            pltpu.SemaphoreType.DMA,
            pltpu.SemaphoreType.DMA,
            pltpu.SemaphoreType.DMA,
        ],
        compiler_params=pltpu.CompilerParams(
            use_tc_tiling_on_sc=False, needs_layout_passes=False),
    )
    def k(tag_hbm, table_hbm, out_hbm, idx_v, q_v, rows_v, ob_v,
          semt0, semt1, semg0, semg1, semo0, semo1):
        wid = lax.axis_index("s") * info.num_cores + lax.axis_index("c")
        base = wid * units_per_w
        iota16 = lax.iota(jnp.int32, 16)
        semt = (semt0, semt1)
        semg = (semg0, semg1)
        semo = (semo0, semo1)

        def tag_slice(u):
            s = u // NBT
            bt = u % NBT
            return tag_hbm.at[s, pl.ds(bt * 128, 128)]

        def compute_q(p):
            for kk in range(8):
                q_v[p, pl.ds(16 * kk, 16)] = idx_v[p, pl.ds(16 * kk, 16)]

        def start_gather(p):
            return pltpu.async_copy(
                table_hbm.at[q_v.at[p]], rows_v.at[p], semg[p])

        c_even = iota16 * 2
        c_odd = c_even + 1
        cteven = lax.shift_right_logical(c_even, 3)
        cieven = lax.bitwise_and(c_even, 7)
        ctodd = lax.shift_right_logical(c_odd, 3)
        ciodd = lax.bitwise_and(c_odd, 7)
        himask = jnp.full((16,), jnp.int32(-65536))
        sh16 = jnp.full((16,), jnp.int32(16))

        def transpose_out(u, p):
            s = u // NBT
            bt = u % NBT
            for b in range(128):
                v16 = rows_v[p, b, :]
                f_even = plsc.bitcast(
                    lax.shift_left(v16, sh16), jnp.float32) * SCALE
                f_odd = plsc.bitcast(
                    lax.bitwise_and(v16, himask), jnp.float32) * SCALE
                b16 = jnp.full((16,), b, jnp.int32)
                plsc.store_scatter(ob_v.at[p], [cteven, cieven, b16], f_even)
                plsc.store_scatter(ob_v.at[p], [ctodd, ciodd, b16], f_odd)
            for ct in range(NCT):
                pltpu.async_copy(
                    ob_v.at[p, ct, :, pl.ds(0, 128)],
                    out_hbm.at[s, ct, bt], semo[p])

        def drain_out(p):
            for ct in range(NCT):
                pltpu.make_async_copy(
                    ob_v.at[p, ct, :, pl.ds(0, 128)],
                    out_hbm.at[0, ct, 0], semo[p]).wait()

        for p in range(2):
            pltpu.sync_copy(tag_slice(base + p), idx_v.at[p])
            compute_q(p)
            start_gather(p)

        def pair(i, carry):
            nxt = base + 2 * ((i + 1) % n_pairs)
            for p in range(2):
                pltpu.async_copy(tag_slice(nxt + p), idx_v.at[p], semt[p])
            for p in range(2):
                u = base + 2 * i + p
                pltpu.make_async_copy(
                    table_hbm.at[q_v.at[p]], rows_v.at[p], semg[p]).wait()

                @pl.when(i > 0)
                def _():
                    drain_out(p)

                transpose_out(u, p)
                pltpu.make_async_copy(
                    tag_slice(nxt + p), idx_v.at[p], semt[p]).wait()
                compute_q(p)
                start_gather(p)
            return carry

        lax.fori_loop(0, n_pairs, pair, 0)

        for p in range(2):
            pltpu.make_async_copy(
                table_hbm.at[q_v.at[p]], rows_v.at[p], semg[p]).wait()
            drain_out(p)

    return k(tag_t, table)


def kernel(tag, table):
    b, s = tag.shape
    tag_t = tag.astype(jnp.int32).T
    table_bits = lax.bitcast_convert_type(
        table.astype(jnp.bfloat16).reshape(table.shape[0], C_DIM // 2, 2),
        jnp.int32)
    out5 = _emb_lookup(tag_t, table_bits)
    return out5.transpose(2, 4, 0, 1, 3).reshape(b, s, C_DIM)


# speedup vs baseline: 1.8193x; 1.8193x over previous
import functools
import math

import jax
import jax.numpy as jnp
from jax import lax
from jax.experimental import pallas as pl
from jax.experimental.pallas import tpu as pltpu
from jax.experimental.pallas import tpu_sc as plsc

C_DIM = 32
SCALE = 1.0 / math.sqrt(C_DIM)


@jax.jit
def _emb_lookup(tag_t, table):
    S, B = tag_t.shape
    NBT = B // 128
    NCT = C_DIM // 8
    info = plsc.get_sparse_core_info()
    nw = info.num_cores * info.num_subcores
    n_units = S * NBT
    units_per_w = n_units // nw
    n_pairs = units_per_w // 2

    mesh = plsc.VectorSubcoreMesh(core_axis_name="c", subcore_axis_name="s")

    @functools.partial(
        pl.kernel,
        mesh=mesh,
        out_type=jax.ShapeDtypeStruct((S, NCT, NBT, 8, 128), jnp.float32),
        scratch_types=[
            pltpu.VMEM((2, 128), jnp.int32),
            pltpu.VMEM((2, 128), jnp.int32),
            pltpu.VMEM((2, 128, C_DIM), jnp.float32),
            pltpu.VMEM((2, NCT, 8, 137), jnp.float32),
            pltpu.SemaphoreType.DMA,
            pltpu.SemaphoreType.DMA,
            pltpu.SemaphoreType.DMA,
            pltpu.SemaphoreType.DMA,
            pltpu.SemaphoreType.DMA,
            pltpu.SemaphoreType.DMA,
        ],
        compiler_params=pltpu.CompilerParams(
            use_tc_tiling_on_sc=False, needs_layout_passes=False),
    )
    def k(tag_hbm, table_hbm, out_hbm, idx_v, q_v, rows_v, ob_v,
          semt0, semt1, semg0, semg1, semo0, semo1):
        wid = lax.axis_index("s") * info.num_cores + lax.axis_index("c")
        base = wid * units_per_w
        iota16 = lax.iota(jnp.int32, 16)
        semt = (semt0, semt1)
        semg = (semg0, semg1)
        semo = (semo0, semo1)

        def tag_slice(u):
            s = u // NBT
            bt = u % NBT
            return tag_hbm.at[s, pl.ds(bt * 128, 128)]

        def compute_q(p):
            for kk in range(8):
                q_v[p, pl.ds(16 * kk, 16)] = idx_v[p, pl.ds(16 * kk, 16)]

        def start_gather(p):
            return pltpu.async_copy(
                table_hbm.at[q_v.at[p]], rows_v.at[p], semg[p])

        ct_lo = lax.shift_right_logical(iota16, 3)
        ci_lo = lax.bitwise_and(iota16, 7)
        ct_hi = lax.shift_right_logical(iota16 + 16, 3)
        ci_hi = lax.bitwise_and(iota16 + 16, 7)

        def transpose_out(u, p):
            s = u // NBT
            bt = u % NBT
            for b in range(128):
                b16 = jnp.full((16,), b, jnp.int32)
                lo = rows_v[p, b, pl.ds(0, 16)] * SCALE
                hi = rows_v[p, b, pl.ds(16, 16)] * SCALE
                plsc.store_scatter(ob_v.at[p], [ct_lo, ci_lo, b16], lo)
                plsc.store_scatter(ob_v.at[p], [ct_hi, ci_hi, b16], hi)
            for ct in range(NCT):
                pltpu.async_copy(
                    ob_v.at[p, ct, :, pl.ds(0, 128)],
                    out_hbm.at[s, ct, bt], semo[p])

        def drain_out(p):
            for ct in range(NCT):
                pltpu.make_async_copy(
                    ob_v.at[p, ct, :, pl.ds(0, 128)],
                    out_hbm.at[0, ct, 0], semo[p]).wait()

        for p in range(2):
            pltpu.sync_copy(tag_slice(base + p), idx_v.at[p])
            compute_q(p)
            start_gather(p)

        def pair(i, carry):
            nxt = base + 2 * ((i + 1) % n_pairs)
            for p in range(2):
                pltpu.async_copy(tag_slice(nxt + p), idx_v.at[p], semt[p])
            for p in range(2):
                u = base + 2 * i + p
                pltpu.make_async_copy(
                    table_hbm.at[q_v.at[p]], rows_v.at[p], semg[p]).wait()

                @pl.when(i > 0)
                def _():
                    drain_out(p)

                transpose_out(u, p)
                pltpu.make_async_copy(
                    tag_slice(nxt + p), idx_v.at[p], semt[p]).wait()
                compute_q(p)
                start_gather(p)
            return carry

        lax.fori_loop(0, n_pairs, pair, 0)

        for p in range(2):
            pltpu.make_async_copy(
                table_hbm.at[q_v.at[p]], rows_v.at[p], semg[p]).wait()
            drain_out(p)

    return k(tag_t, table)


def kernel(tag, table):
    b, s = tag.shape
    tag_t = tag.astype(jnp.int32).T
    out5 = _emb_lookup(tag_t, table)
    return out5.transpose(2, 4, 0, 1, 3).reshape(b, s, C_DIM)


# speedup vs baseline: 1.8833x; 1.0352x over previous
import functools
import math

import jax
import jax.numpy as jnp
from jax import lax
from jax.experimental import pallas as pl
from jax.experimental.pallas import tpu as pltpu
from jax.experimental.pallas import tpu_sc as plsc

C_DIM = 32
SCALE = 1.0 / math.sqrt(C_DIM)


@jax.jit
def _emb_lookup(tag_t, table):
    S, B = tag_t.shape
    NBT = B // 128
    NCT = C_DIM // 8
    info = plsc.get_sparse_core_info()
    nw = info.num_cores * info.num_subcores
    n_units = S * NBT
    units_per_w = n_units // nw
    n_pairs = units_per_w // 2

    mesh = plsc.VectorSubcoreMesh(core_axis_name="c", subcore_axis_name="s")

    @functools.partial(
        pl.kernel,
        mesh=mesh,
        out_type=jax.ShapeDtypeStruct((S, NCT, NBT, 8, 128), jnp.float32),
        scratch_types=[
            pltpu.VMEM((2, 128), jnp.int32),
            pltpu.VMEM((2, 128), jnp.int32),
            pltpu.VMEM((2, 128, C_DIM), jnp.float32),
            pltpu.VMEM((2, NCT, 8, 137), jnp.float32),
            pltpu.SemaphoreType.DMA,
            pltpu.SemaphoreType.DMA,
            pltpu.SemaphoreType.DMA,
            pltpu.SemaphoreType.DMA,
            pltpu.SemaphoreType.DMA,
            pltpu.SemaphoreType.DMA,
        ],
        compiler_params=pltpu.CompilerParams(
            use_tc_tiling_on_sc=False, needs_layout_passes=False),
    )
    def k(tag_hbm, table_hbm, out_hbm, idx_v, q_v, rows_v, ob_v,
          semt0, semt1, semg0, semg1, semo0, semo1):
        wid = lax.axis_index("s") * info.num_cores + lax.axis_index("c")
        base = wid * units_per_w
        iota16 = lax.iota(jnp.int32, 16)
        semt = (semt0, semt1)
        semg = (semg0, semg1)
        semo = (semo0, semo1)

        def tag_slice(u):
            s = u // NBT
            bt = u % NBT
            return tag_hbm.at[s, pl.ds(bt * 128, 128)]

        def compute_q(p):
            for kk in range(8):
                q_v[p, pl.ds(16 * kk, 16)] = idx_v[p, pl.ds(16 * kk, 16)]

        def start_gather(p):
            return pltpu.async_copy(
                table_hbm.at[q_v.at[p]], rows_v.at[p], semg[p])

        def transpose_out(u, p):
            s = u // NBT
            bt = u % NBT
            for half in range(2):
                c0 = 16 * half
                ct16 = lax.shift_right_logical(iota16 + c0, 3)
                ci16 = lax.bitwise_and(iota16 + c0, 7)
                for b in range(128):
                    val = rows_v[p, b, pl.ds(c0, 16)] * SCALE
                    b16 = jnp.full((16,), b, jnp.int32)
                    plsc.store_scatter(ob_v.at[p], [ct16, ci16, b16], val)
            pltpu.async_copy(
                ob_v.at[p, :, :, pl.ds(0, 128)],
                out_hbm.at[s, :, bt], semo[p])

        def drain_out(p):
            pltpu.make_async_copy(
                ob_v.at[p, :, :, pl.ds(0, 128)],
                out_hbm.at[0, :, 0], semo[p]).wait()

        for p in range(2):
            pltpu.sync_copy(tag_slice(base + p), idx_v.at[p])
            compute_q(p)
            start_gather(p)

        def pair(i, carry):
            nxt = base + 2 * ((i + 1) % n_pairs)
            for p in range(2):
                pltpu.async_copy(tag_slice(nxt + p), idx_v.at[p], semt[p])
            for p in range(2):
                u = base + 2 * i + p
                pltpu.make_async_copy(
                    table_hbm.at[q_v.at[p]], rows_v.at[p], semg[p]).wait()

                @pl.when(i > 0)
                def _():
                    drain_out(p)

                transpose_out(u, p)
                pltpu.make_async_copy(
                    tag_slice(nxt + p), idx_v.at[p], semt[p]).wait()
                compute_q(p)
                start_gather(p)
            return carry

        lax.fori_loop(0, n_pairs, pair, 0)

        for p in range(2):
            pltpu.make_async_copy(
                table_hbm.at[q_v.at[p]], rows_v.at[p], semg[p]).wait()
            drain_out(p)

    return k(tag_t, table)


def kernel(tag, table):
    b, s = tag.shape
    tag_t = tag.astype(jnp.int32).T
    out5 = _emb_lookup(tag_t, table)
    return out5.transpose(2, 4, 0, 1, 3).reshape(b, s, C_DIM)
